# baseline (speedup 1.0000x reference)
import functools

import jax
import jax.numpy as jnp
from jax import lax
from jax.experimental import pallas as pl
from jax.experimental.pallas import tpu as pltpu
from jax.experimental.pallas import tpu_sc as plsc

_N = 16384
_D = 2048
_NW = 32
_PIECE = 16
_NBUF = 2


def _sc_copy(x_hbm, y_hbm, out_hbm, bufs, sems):
    wid = lax.axis_index("s") * 2 + lax.axis_index("c")
    rows_per_w = _N // _NW
    base = wid * rows_per_w
    n_pieces = rows_per_w // _PIECE

    def in_cp(p, slot):
        return pltpu.make_async_copy(
            y_hbm.at[pl.ds(base + p * _PIECE, _PIECE), :], bufs.at[slot],
            sems.at[slot])

    def out_cp(p, slot):
        return pltpu.make_async_copy(
            bufs.at[slot], out_hbm.at[pl.ds(base + p * _PIECE, _PIECE), :],
            sems.at[slot])

    in_cp(0, 0).start()
    in_cp(1, 1).start()

    def body(p, _):
        slot = lax.rem(p, _NBUF)
        in_cp(p, slot).wait()
        out_cp(p, slot).start()
        out_cp(p, slot).wait()

        @pl.when(p + _NBUF < n_pieces)
        def _prefetch():
            in_cp(p + _NBUF, slot).start()
        return _

    lax.fori_loop(0, n_pieces, body, 0)


def kernel(x, y):
    n, d = y.shape
    mesh = plsc.VectorSubcoreMesh(core_axis_name="c", subcore_axis_name="s")
    k = functools.partial(
        pl.kernel,
        out_type=jax.ShapeDtypeStruct((n, d), y.dtype),
        mesh=mesh,
        scratch_types=[
            pltpu.VMEM((_NBUF, _PIECE, _D), jnp.float32),
            pltpu.SemaphoreType.DMA((_NBUF,)),
        ],
    )(_sc_copy)
    return k(x.astype(jnp.int32), y)
